# baseline (speedup 1.0000x reference)
import functools

import jax
import jax.numpy as jnp
from jax import lax
from jax.experimental import pallas as pl
from jax.experimental.pallas import tpu as pltpu
from jax.experimental.pallas import tpu_sc as plsc

_NS = 16
_PAD = 16


def _sc_gather_rows(emb_table, input_ids, hid):
    b, nw = input_ids.shape
    n_flat = b * nw
    rows_per_w = n_flat // _NS

    mesh = plsc.VectorSubcoreMesh(
        core_axis_name="c", subcore_axis_name="s", num_cores=1)

    @functools.partial(
        pl.kernel,
        mesh=mesh,
        out_type=jax.ShapeDtypeStruct((n_flat, hid), jnp.float32),
        scratch_types=[
            pltpu.VMEM((rows_per_w,), jnp.int32),
            pltpu.VMEM((rows_per_w, hid), jnp.float32),
            pltpu.SemaphoreType.DMA,
        ],
    )
    def gather_kernel(table_hbm, idx_hbm, out_hbm, idx_v, rows_v, sem):
        wid = lax.axis_index("s")
        base = wid * rows_per_w
        pltpu.sync_copy(idx_hbm.at[base // nw, pl.ds(base % nw, rows_per_w)],
                        idx_v)
        pltpu.async_copy(table_hbm.at[idx_v], rows_v, sem).wait()
        pltpu.sync_copy(rows_v, out_hbm.at[pl.ds(base, rows_per_w)])

    return gather_kernel(emb_table, input_ids)


def _tc_kernel(stride_ref, words_lo_ref, words_hi_ref, zrow_ref, bbox_ref,
               proj_ref, out_ref):
    bh = pl.num_programs(0) // 2
    words = jnp.where(pl.program_id(0) < bh,
                      words_lo_ref[...], words_hi_ref[...])
    _tc_body(stride_ref, words, zrow_ref, bbox_ref, proj_ref, out_ref)


def _tc_body(stride_ref, words, zrow_ref, bbox_ref, proj_ref, out_ref):
    nw = words.shape[0]
    h, w = out_ref.shape[2], out_ref.shape[3]
    ngrp = nw // 16
    kdim = nw + 8

    rows_ext = jnp.concatenate([words, zrow_ref[...]], axis=0)
    pt = lax.dot_general(
        proj_ref[...], rows_ext,
        (((0,), (1,)), ((), ())),
        preferred_element_type=jnp.float32,
    )

    stride_f = stride_ref[0, 0].astype(jnp.float32)
    bb = jnp.rint(bbox_ref[0] / stride_f)
    ws_c, we_c = bb[:, 0:1], bb[:, 2:3]
    hs_c, he_c = bb[:, 1:2], bb[:, 3:4]

    n_col = lax.broadcasted_iota(jnp.int32, (nw, 1), 0)
    w_col = (jnp.int32(1) << (n_col & 15)).astype(jnp.float32)
    yy = lax.broadcasted_iota(jnp.int32, (nw, h), 1).astype(jnp.float32)
    yt_mat = jnp.where((yy >= hs_c) & (yy < he_c), w_col, 0.0)

    jlane = lax.broadcasted_iota(jnp.int32, (nw, ngrp * w), 1)
    xf = (jlane & (w - 1)).astype(jnp.float32)
    in_grp = (n_col >> 4) == (jlane >> 7)
    x_mat = jnp.where((xf >= ws_c) & (xf < we_c) & in_grp, 1.0, 0.0)

    s = lax.dot_general(
        yt_mat, x_mat,
        (((0,), (0,)), ((), ())),
        preferred_element_type=jnp.float32,
    )
    e = (lax.bitcast_convert_type(s, jnp.int32) >> 23) - 127
    cand = e + (lax.broadcasted_iota(jnp.int32, (h, ngrp * w), 1) >> 7) * 16
    best = cand[:, 0:w]
    for g in range(1, ngrp):
        best = jnp.maximum(best, cand[:, g * w:(g + 1) * w])
    sel = jnp.where(best >= 0, best, nw)

    sel_flat = jnp.reshape(sel, (1, h * w))
    k_iota = lax.broadcasted_iota(jnp.int32, (kdim, h * w), 0)
    oh = (k_iota == sel_flat).astype(jnp.float32)
    out_t = jnp.dot(pt, oh, preferred_element_type=jnp.float32)
    out_ref[0] = jnp.reshape(out_t, (out_t.shape[0], h, w))


def kernel(img, input_ids, bbox, emb_table, proj_w, stride):
    b, _, img_h, img_w = img.shape
    h, w = img_h // 4, img_w // 4
    nw = input_ids.shape[1]
    hid = emb_table.shape[1]
    edim = proj_w.shape[1]
    n_flat = b * nw

    stride_arr = jnp.reshape(jnp.asarray(stride, jnp.float32), (1, 1))
    bh = b // 2

    rows_lo = _sc_gather_rows(emb_table, input_ids[:bh], hid)
    rows_hi = _sc_gather_rows(emb_table, input_ids[bh:], hid)

    return pl.pallas_call(
        _tc_kernel,
        grid=(b,),
        in_specs=[
            pl.BlockSpec(memory_space=pltpu.SMEM),
            pl.BlockSpec((nw, hid), lambda i: (jnp.minimum(i, bh - 1), 0)),
            pl.BlockSpec((nw, hid), lambda i: (jnp.maximum(i - bh, 0), 0)),
            pl.BlockSpec((8, hid), lambda i: (0, 0)),
            pl.BlockSpec((1, nw, 4), lambda i: (i, 0, 0)),
            pl.BlockSpec((hid, edim), lambda i: (0, 0)),
        ],
        out_specs=pl.BlockSpec((1, edim, h, w), lambda i: (i, 0, 0, 0)),
        out_shape=jax.ShapeDtypeStruct((b, edim, h, w), jnp.float32),
    )(stride_arr, rows_lo, rows_hi, emb_table, bbox, proj_w)


# speedup vs baseline: 1.0524x; 1.0524x over previous
import functools

import jax
import jax.numpy as jnp
from jax import lax
from jax.experimental import pallas as pl
from jax.experimental.pallas import tpu as pltpu
from jax.experimental.pallas import tpu_sc as plsc

_NS = 16
_PAD = 16


def _sc_gather_rows(emb_table, input_ids, hid):
    b, nw = input_ids.shape
    n_flat = b * nw
    rows_per_w = n_flat // _NS

    mesh = plsc.VectorSubcoreMesh(
        core_axis_name="c", subcore_axis_name="s", num_cores=1)

    @functools.partial(
        pl.kernel,
        mesh=mesh,
        out_type=jax.ShapeDtypeStruct((n_flat, hid), jnp.float32),
        scratch_types=[
            pltpu.VMEM((rows_per_w,), jnp.int32),
            pltpu.VMEM((rows_per_w, hid), jnp.float32),
            pltpu.SemaphoreType.DMA,
        ],
    )
    def gather_kernel(table_hbm, idx_hbm, out_hbm, idx_v, rows_v, sem):
        wid = lax.axis_index("s")
        base = wid * rows_per_w
        pltpu.sync_copy(idx_hbm.at[base // nw, pl.ds(base % nw, rows_per_w)],
                        idx_v)
        pltpu.async_copy(table_hbm.at[idx_v], rows_v, sem).wait()
        pltpu.sync_copy(rows_v, out_hbm.at[pl.ds(base, rows_per_w)])

    return gather_kernel(emb_table, input_ids)


def _tc_body(stride_ref, words_ref, zrow_ref, bbox_ref, proj_ref, h, w):
    nw = words_ref.shape[0]
    ngrp = nw // 16
    kdim = nw + 8

    rows_ext = jnp.concatenate([words_ref[...], zrow_ref[...]], axis=0)
    pt = lax.dot_general(
        proj_ref[...], rows_ext,
        (((0,), (1,)), ((), ())),
        preferred_element_type=jnp.float32,
    )

    stride_f = stride_ref[0, 0].astype(jnp.float32)
    bb = jnp.rint(bbox_ref[0] / stride_f)
    ws_c, we_c = bb[:, 0:1], bb[:, 2:3]
    hs_c, he_c = bb[:, 1:2], bb[:, 3:4]

    n_col = lax.broadcasted_iota(jnp.int32, (nw, 1), 0)
    w_col = (jnp.int32(1) << (n_col & 15)).astype(jnp.float32)
    yy = lax.broadcasted_iota(jnp.int32, (nw, h), 1).astype(jnp.float32)
    yt_mat = jnp.where((yy >= hs_c) & (yy < he_c), w_col, 0.0)

    jlane = lax.broadcasted_iota(jnp.int32, (nw, ngrp * w), 1)
    xf = (jlane & (w - 1)).astype(jnp.float32)
    in_grp = (n_col >> 4) == (jlane >> 7)
    x_mat = jnp.where((xf >= ws_c) & (xf < we_c) & in_grp, 1.0, 0.0)

    s = lax.dot_general(
        yt_mat, x_mat,
        (((0,), (0,)), ((), ())),
        preferred_element_type=jnp.float32,
    )
    e = (lax.bitcast_convert_type(s, jnp.int32) >> 23) - 127
    cand = e + (lax.broadcasted_iota(jnp.int32, (h, ngrp * w), 1) >> 7) * 16
    best = cand[:, 0:w]
    for g in range(1, ngrp):
        best = jnp.maximum(best, cand[:, g * w:(g + 1) * w])
    sel = jnp.where(best >= 0, best, nw)

    sel_flat = jnp.reshape(sel, (1, h * w))
    k_iota = lax.broadcasted_iota(jnp.int32, (kdim, h * w), 0)
    oh = (k_iota == sel_flat).astype(jnp.float32)
    out_t = jnp.dot(pt, oh, preferred_element_type=jnp.float32)
    return jnp.reshape(out_t, (out_t.shape[0], h, w))


def _make_tc_half(b, bh, b0, nw, hid, edim, h, w, aliased):
    def body(stride_ref, words_ref, zrow_ref, bbox_ref, proj_ref,
             *rest):
        out_ref, scratch_ref, sems = rest[-3], rest[-2], rest[-1]
        i = pl.program_id(0)
        out_t = _tc_body(stride_ref, words_ref, zrow_ref, bbox_ref, proj_ref,
                         h, w)
        scratch_ref[i] = out_t
        pltpu.make_async_copy(
            scratch_ref.at[i], out_ref.at[b0 + i], sems.at[i]).start()

        @pl.when(i == bh - 1)
        def _():
            for j in range(bh):
                pltpu.make_async_copy(
                    scratch_ref.at[j], out_ref.at[b0 + j], sems.at[j]).wait()

    in_specs = [
        pl.BlockSpec(memory_space=pltpu.SMEM),
        pl.BlockSpec((nw, hid), lambda i: (i, 0)),
        pl.BlockSpec((8, hid), lambda i: (0, 0)),
        pl.BlockSpec((1, nw, 4), lambda i: (i + b0, 0, 0)),
        pl.BlockSpec((hid, edim), lambda i: (0, 0)),
    ]
    if aliased:
        in_specs.append(pl.BlockSpec(memory_space=pl.ANY))
    return pl.pallas_call(
        body,
        grid=(bh,),
        in_specs=in_specs,
        out_specs=pl.BlockSpec(memory_space=pl.ANY),
        out_shape=jax.ShapeDtypeStruct((b, edim, h, w), jnp.float32),
        scratch_shapes=[
            pltpu.VMEM((bh, edim, h, w), jnp.float32),
            pltpu.SemaphoreType.DMA((bh,)),
        ],
        input_output_aliases={5: 0} if aliased else {},
    )


def kernel(img, input_ids, bbox, emb_table, proj_w, stride):
    b, _, img_h, img_w = img.shape
    h, w = img_h // 4, img_w // 4
    nw = input_ids.shape[1]
    hid = emb_table.shape[1]
    edim = proj_w.shape[1]
    bh = b // 2

    rows_lo = _sc_gather_rows(emb_table, input_ids[:bh], hid)
    rows_hi = _sc_gather_rows(emb_table, input_ids[bh:], hid)
    stride_arr = jnp.reshape(jnp.asarray(stride, jnp.float32), (1, 1))

    half = _make_tc_half(b, bh, 0, nw, hid, edim, h, w, aliased=False)(
        stride_arr, rows_lo, emb_table, bbox, proj_w)
    return _make_tc_half(b, bh, bh, nw, hid, edim, h, w, aliased=True)(
        stride_arr, rows_hi, emb_table, bbox, proj_w, half)


# speedup vs baseline: 1.1510x; 1.0937x over previous
import functools

import jax
import jax.numpy as jnp
from jax import lax
from jax.experimental import pallas as pl
from jax.experimental.pallas import tpu as pltpu
from jax.experimental.pallas import tpu_sc as plsc

_NS = 16
_PAD = 16


def _sc_gather_rows(emb_table, input_ids, hid):
    b, nw = input_ids.shape
    n_flat = b * nw
    rows_per_w = n_flat // _NS

    mesh = plsc.VectorSubcoreMesh(
        core_axis_name="c", subcore_axis_name="s", num_cores=1)

    @functools.partial(
        pl.kernel,
        mesh=mesh,
        out_type=jax.ShapeDtypeStruct((n_flat, hid), jnp.float32),
        scratch_types=[
            pltpu.VMEM((rows_per_w,), jnp.int32),
            pltpu.VMEM((rows_per_w, hid), jnp.float32),
            pltpu.SemaphoreType.DMA,
            pltpu.SemaphoreType.DMA,
            pltpu.SemaphoreType.DMA,
            pltpu.SemaphoreType.DMA,
        ],
    )
    def gather_kernel(table_hbm, idx_hbm, out_hbm, idx_v, rows_v,
                      sem_a, sem_b, sem_oa, sem_ob):
        wid = lax.axis_index("s")
        base = wid * rows_per_w
        half = rows_per_w // 2
        pltpu.sync_copy(idx_hbm.at[base // nw, pl.ds(base % nw, rows_per_w)],
                        idx_v)
        g_a = pltpu.async_copy(table_hbm.at[idx_v.at[pl.ds(0, half)]],
                               rows_v.at[pl.ds(0, half)], sem_a)
        g_b = pltpu.async_copy(table_hbm.at[idx_v.at[pl.ds(half, half)]],
                               rows_v.at[pl.ds(half, half)], sem_b)
        g_a.wait()
        o_a = pltpu.async_copy(rows_v.at[pl.ds(0, half)],
                               out_hbm.at[pl.ds(base, half)], sem_oa)
        g_b.wait()
        o_b = pltpu.async_copy(rows_v.at[pl.ds(half, half)],
                               out_hbm.at[pl.ds(base + half, half)], sem_ob)
        o_a.wait()
        o_b.wait()

    return gather_kernel(emb_table, input_ids)


def _tc_kernel(stride_ref, words_ref, zrow_ref, bbox_ref, proj_ref, out_ref):
    nw = words_ref.shape[0]
    h, w = out_ref.shape[2], out_ref.shape[3]
    ngrp = nw // 16
    kdim = nw + 8

    rows_ext = jnp.concatenate([words_ref[...], zrow_ref[...]], axis=0)
    pt = lax.dot_general(
        proj_ref[...], rows_ext,
        (((0,), (1,)), ((), ())),
        preferred_element_type=jnp.float32,
    )

    stride_f = stride_ref[0, 0].astype(jnp.float32)
    bb = jnp.rint(bbox_ref[0] / stride_f)
    ws_c, we_c = bb[:, 0:1], bb[:, 2:3]
    hs_c, he_c = bb[:, 1:2], bb[:, 3:4]

    n_col = lax.broadcasted_iota(jnp.int32, (nw, 1), 0)
    w_col = (jnp.int32(1) << (n_col & 15)).astype(jnp.float32)
    yy = lax.broadcasted_iota(jnp.int32, (nw, h), 1).astype(jnp.float32)
    yt_mat = jnp.where((yy >= hs_c) & (yy < he_c), w_col, 0.0)

    jlane = lax.broadcasted_iota(jnp.int32, (nw, ngrp * w), 1)
    xf = (jlane & (w - 1)).astype(jnp.float32)
    in_grp = (n_col >> 4) == (jlane >> 7)
    x_mat = jnp.where((xf >= ws_c) & (xf < we_c) & in_grp, 1.0, 0.0)

    s = lax.dot_general(
        yt_mat, x_mat,
        (((0,), (0,)), ((), ())),
        preferred_element_type=jnp.float32,
    )
    e = (lax.bitcast_convert_type(s, jnp.int32) >> 23) - 127
    cand = e + (lax.broadcasted_iota(jnp.int32, (h, ngrp * w), 1) >> 7) * 16
    best = cand[:, 0:w]
    for g in range(1, ngrp):
        best = jnp.maximum(best, cand[:, g * w:(g + 1) * w])
    sel = jnp.where(best >= 0, best, nw)

    sel_flat = jnp.reshape(sel, (1, h * w))
    k_iota = lax.broadcasted_iota(jnp.int32, (kdim, h * w), 0)
    oh = (k_iota == sel_flat).astype(jnp.float32)
    out_t = jnp.dot(pt, oh, preferred_element_type=jnp.float32)
    out_ref[0] = jnp.reshape(out_t, (out_t.shape[0], h, w))


def kernel(img, input_ids, bbox, emb_table, proj_w, stride):
    b, _, img_h, img_w = img.shape
    h, w = img_h // 4, img_w // 4
    nw = input_ids.shape[1]
    hid = emb_table.shape[1]
    edim = proj_w.shape[1]
    n_flat = b * nw

    rows = _sc_gather_rows(emb_table, input_ids, hid)
    stride_arr = jnp.reshape(jnp.asarray(stride, jnp.float32), (1, 1))

    return pl.pallas_call(
        _tc_kernel,
        grid=(b,),
        in_specs=[
            pl.BlockSpec(memory_space=pltpu.SMEM),
            pl.BlockSpec((nw, hid), lambda i: (i, 0)),
            pl.BlockSpec((8, hid), lambda i: (0, 0)),
            pl.BlockSpec((1, nw, 4), lambda i: (i, 0, 0)),
            pl.BlockSpec((hid, edim), lambda i: (0, 0)),
        ],
        out_specs=pl.BlockSpec((1, edim, h, w), lambda i: (i, 0, 0, 0)),
        out_shape=jax.ShapeDtypeStruct((b, edim, h, w), jnp.float32),
    )(stride_arr, rows, emb_table, bbox, proj_w)
